# baseline (speedup 1.0000x reference)
import jax
import jax.numpy as jnp
from jax import lax
from jax.experimental import pallas as pl
from jax.experimental.pallas import tpu as pltpu

_B, _H, _W, _D = 32, 32, 32, 1024
_K = 9
_T = 9
_G = 2
_R = _G * _H * _W


def _softmax_head(x, w2, b):
    z = jnp.dot(x, w2, preferred_element_type=jnp.float32)
    zero_row = jnp.zeros_like(z[:1])
    z_by_dj = {
        0: z,
        1: jnp.concatenate([z[1:], zero_row], 0),
        -1: jnp.concatenate([zero_row, z[:-1]], 0),
    }
    row = lax.broadcasted_iota(jnp.int32, (_R, _T * _K), 0)
    w_of_r = row & (_W - 1)
    h_of_r = (row // _W) & (_H - 1)
    lane = lax.broadcasted_iota(jnp.int32, (_R, _T * _K), 1)
    acc = jnp.zeros((_R, _T * _K), jnp.float32)
    for t in range(_T):
        di, dj = t // 3 - 1, t % 3 - 1
        zt = z_by_dj[dj]
        if di == 1:
            zt = jnp.concatenate([zt[_W:], jnp.zeros_like(zt[:_W])], 0)
        elif di == -1:
            zt = jnp.concatenate([jnp.zeros_like(zt[:_W]), zt[:-_W]], 0)
        ok = (lane >= t * _K) & (lane < (t + 1) * _K)
        if dj == 1:
            ok = ok & (w_of_r < _W - 1)
        elif dj == -1:
            ok = ok & (w_of_r > 0)
        if di == 1:
            ok = ok & (h_of_r < _H - 1)
        elif di == -1:
            ok = ok & (h_of_r > 0)
        acc = acc + jnp.where(ok, zt, 0.0)
    g = (lax.broadcasted_iota(jnp.int32, (_T * _K, _K), 0) % _K
         == lax.broadcasted_iota(jnp.int32, (_T * _K, _K), 1))
    logits = jnp.dot(acc, g.astype(jnp.float32),
                     preferred_element_type=jnp.float32)
    logits = logits + b
    m = jnp.max(logits, axis=-1, keepdims=True)
    e = jnp.exp(logits - m)
    return e / jnp.sum(e, axis=-1, keepdims=True)


def _affinity_kernel(xa_ref, xb_ref, w2_ref, b_ref, oa_ref, ob_ref):
    w2 = w2_ref[...]
    b = b_ref[...]
    oa_ref[...] = _softmax_head(xa_ref[...].reshape(_R, _D), w2, b).reshape(
        _G, _H * _W, _K)
    ob_ref[...] = _softmax_head(xb_ref[...].reshape(_R, _D), w2, b).reshape(
        _G, _H * _W, _K)


def kernel(tok2d, dw_w, pw_w, pw_b):
    x = tok2d.reshape(_B, _H * _W, _D)
    dw2 = dw_w.reshape(_D, _T)
    pw2 = pw_w.reshape(_K, _D).T
    w2 = (dw2[:, :, None] * pw2[:, None, :]).reshape(_D, _T * _K)
    b2 = pw_b.reshape(1, _K)
    qa, qb = pl.pallas_call(
        _affinity_kernel,
        grid=(_B // (2 * _G),),
        in_specs=[
            pl.BlockSpec((_G, _H * _W, _D), lambda b: (2 * b, 0, 0)),
            pl.BlockSpec((_G, _H * _W, _D), lambda b: (2 * b + 1, 0, 0)),
            pl.BlockSpec((_D, _T * _K), lambda b: (0, 0)),
            pl.BlockSpec((1, _K), lambda b: (0, 0)),
        ],
        out_specs=[
            pl.BlockSpec((_G, _H * _W, _K), lambda b: (b, 0, 0)),
            pl.BlockSpec((_G, _H * _W, _K), lambda b: (b, 0, 0)),
        ],
        out_shape=[
            jax.ShapeDtypeStruct((_B // 2, _H * _W, _K), jnp.float32),
            jax.ShapeDtypeStruct((_B // 2, _H * _W, _K), jnp.float32),
        ],
        compiler_params=pltpu.CompilerParams(
            dimension_semantics=("parallel",),
        ),
    )(x, x, w2, b2)
    q = jnp.concatenate(
        [qa.reshape(_B // (2 * _G), _G, _H * _W, _K),
         qb.reshape(_B // (2 * _G), _G, _H * _W, _K)], axis=1)
    return q.reshape(_B, _H, _W, _K)
